# baseline (speedup 1.0000x reference)
import jax
import jax.numpy as jnp
from jax import lax
from jax.experimental import pallas as pl
from jax.experimental.pallas import tpu as pltpu
from jax.experimental.pallas import tpu_sc as plsc

N = 32768
K = 1024
ROWS = 128
NT = 16
CH = N // NT
VPC = CH // 16
TRASH = 16


def _topk_select_body(scores_hbm, out_hbm,
                      score_v, km_v, cand_v, cand2_v, hist_v, hist2_v,
                      lk_v, li_v, ei_v, dref_v, misc_v, cnt_v, skey_v,
                      sidx_v, skey2_v, sidx2_v,
                      shistA_s, shistB_s, shistC_s, cnts_s, bc_s, selk_s,
                      seli_s):
    c = lax.axis_index("c")
    t = lax.axis_index("s")
    iota = lax.iota(jnp.int32, 16)
    zero16 = jnp.zeros((16,), jnp.int32)
    i32 = jnp.int32
    u32 = jnp.uint32

    def _lane(v, i):
        return jnp.sum(jnp.where(iota == i, v, 0))

    def _clear_hist(nreg):
        def b(vv, _):
            for u in range(8):
                hist_v[pl.ds(128 * vv + 16 * u, 16)] = zero16
            return 0
        lax.fori_loop(0, nreg // 8, b, 0)

    rc0, _ = plsc.scan_count(zero16)
    bias = jnp.min(rc0.astype(i32))

    pltpu.sync_copy(scores_hbm.at[pl.ds(t * CH, CH)], score_v)
    _clear_hist(VPC)
    pltpu.sync_copy(hist_v.at[pl.ds(0, 128)], shistA_s.at[pl.ds(t * 128, 128)])
    pltpu.sync_copy(hist_v.at[pl.ds(0, 128)], shistB_s.at[pl.ds(t * 128, 128)])
    pltpu.sync_copy(hist_v.at[pl.ds(0, 64)], shistC_s.at[pl.ds(t * 64, 64)])

    def km_body(vv, _):
        for u in range(4):
            o = 64 * vv + 16 * u
            f = score_v[pl.ds(o, 16)]
            b = plsc.bitcast(f, u32)
            neg = (b & u32(0x80000000)) != u32(0)
            m = jnp.where(neg, ~b, b | u32(0x80000000))
            km_v[pl.ds(o, 16)] = ~m
        return 0
    lax.fori_loop(0, VPC // 4, km_body, 0)
    plsc.subcore_barrier()

    def local_hist(src_ref, nvreg4, shift, mask):
        def b(vv, _):
            for u in range(4):
                kv = src_ref[pl.ds(64 * vv + 16 * u, 16)]
                d = ((kv >> u32(shift)) & u32(mask)).astype(i32)
                rc, lastm = plsc.scan_count(d)
                plsc.addupdate_scatter(hist_v, [d], rc.astype(i32) - bias + 1,
                                       mask=lastm)
            return 0
        lax.fori_loop(0, nvreg4, b, 0)

    def scatter_add_hist(dst_s, nchunk):
        def b(j, _):
            for u in range(8):
                dref_v[0, pl.ds(16 * u, 16)] = iota + (16 * u + 128 * j)
            pltpu.sync_copy(hist_v.at[pl.ds(128 * j, 128)],
                            dst_s.at[dref_v.at[0]], add=True)
            return 0
        lax.fori_loop(0, nchunk, b, 0)

    def scan_and_publish(src_s, nvreg, target):
        pltpu.sync_copy(src_s, hist_v.at[pl.ds(0, 16 * nvreg)])

        def b(vv, carry):
            cum, vstar, cbef = carry
            for u in range(4):
                h = hist_v[pl.ds(64 * vv + 16 * u, 16)]
                s = jnp.sum(h)
                ncum = cum + s
                cross = (cum < target) & (ncum >= target)
                vstar = jnp.where(cross, 4 * vv + u, vstar)
                cbef = jnp.where(cross, cum, cbef)
                cum = ncum
            return (cum, vstar, cbef)
        _, vstar, cb = lax.fori_loop(0, nvreg // 4, b,
                                     (i32(0), i32(0), i32(0)))
        h = hist_v[pl.ds(16 * vstar, 16)]
        ch = plsc.cumsum(h) + cb
        hitv = ch >= target
        pos = jnp.max(plsc.all_reduce_ffs(hitv).astype(i32))
        bst = 16 * vstar + pos
        cbef = _lane(ch - h, pos)
        misc_v[...] = jnp.where(iota == 0, bst, jnp.where(iota == 1, cbef, 0))
        pltpu.sync_copy(misc_v, bc_s)

    def read_bc():
        pltpu.sync_copy(bc_s, misc_v)
        v = misc_v[...]
        return _lane(v, 0), _lane(v, 1)

    def compact_cands(src_ref, dst_ref, nvreg4, shift, mask, want):
        def b(vv, off):
            for u in range(4):
                kv = src_ref[pl.ds(64 * vv + 16 * u, 16)]
                mk = ((kv >> u32(shift)) & u32(mask)).astype(i32) == want
                plsc.store_compressed(dst_ref.at[pl.ds(off, 16)], kv, mask=mk)
                off = off + jnp.sum(mk.astype(i32))
            return off
        n = lax.fori_loop(0, nvreg4, b, i32(0))
        sent = jnp.full((16,), 0xFFFFFFFF, u32)
        for u in range(4):
            dst_ref[pl.ds(n + 16 * u, 16)] = sent
        return n

    _clear_hist(VPC)
    local_hist(km_v, VPC // 4, 21, 0x7FF)
    scatter_add_hist(shistA_s, 16)
    plsc.subcore_barrier()

    @pl.when(t == 0)
    def _():
        scan_and_publish(shistA_s, VPC, i32(K))
    plsc.subcore_barrier()
    b1, c1 = read_bc()
    r1 = i32(K - 1) - c1

    nc1 = compact_cands(km_v, cand_v, VPC // 4, 21, 0x7FF, b1)
    ncv1 = (nc1 + 63) // 64
    _clear_hist(VPC)
    local_hist(cand_v, ncv1, 10, 0x7FF)
    scatter_add_hist(shistB_s, 16)
    plsc.subcore_barrier()

    @pl.when(t == 0)
    def _():
        scan_and_publish(shistB_s, VPC, r1 + 1)
    plsc.subcore_barrier()
    b2, c2 = read_bc()
    r2 = r1 - c2

    nc2 = compact_cands(cand_v, cand2_v, ncv1, 10, 0x7FF, b2)
    ncv2 = (nc2 + 63) // 64
    _clear_hist(64)
    local_hist(cand2_v, ncv2, 0, 0x3FF)
    scatter_add_hist(shistC_s, 8)
    plsc.subcore_barrier()

    @pl.when(t == 0)
    def _():
        scan_and_publish(shistC_s, 64, r2 + 1)
    plsc.subcore_barrier()
    b3, c3 = read_bc()

    T = ((b1.astype(u32) << 21) | (b2.astype(u32) << 10) | b3.astype(u32))
    L = c1 + c2 + c3
    mneed = i32(K) - L

    def comp2(vv, carry):
        offl, offe = carry
        for u in range(4):
            o = 64 * vv + 16 * u
            kv = km_v[pl.ds(o, 16)]
            gi = iota + (o + CH * t)
            ml = kv < T
            me = kv == T
            plsc.store_compressed(lk_v.at[pl.ds(offl, 16)], kv, mask=ml)
            plsc.store_compressed(li_v.at[pl.ds(offl, 16)], gi, mask=ml)
            plsc.store_compressed(ei_v.at[pl.ds(offe, 16)], gi, mask=me)
            offl = offl + jnp.sum(ml.astype(i32))
            offe = offe + jnp.sum(me.astype(i32))
        return (offl, offe)
    cntl, cnte = lax.fori_loop(0, VPC // 4, comp2, (i32(0), i32(0)))

    misc_v[...] = jnp.where(iota == 0, cntl, jnp.where(iota == 1, cnte, 0))
    pltpu.sync_copy(misc_v, cnts_s.at[t])
    plsc.subcore_barrier()

    pltpu.sync_copy(cnts_s, cnt_v)
    lessc = plsc.load_gather(cnt_v, [iota, zero16])
    eqc = plsc.load_gather(cnt_v, [iota, zero16 + 1])
    cl = plsc.cumsum(lessc)
    ce = plsc.cumsum(eqc)
    offl_me = _lane(cl - lessc, t)
    offe_me = _lane(ce - eqc, t)

    tk = plsc.bitcast(jnp.zeros((16,), u32) + T, i32)
    for u in range(8):
        skey_v[pl.ds(16 * u, 16)] = tk

    def put_less(j, _):
        for u in range(8):
            pos = iota + (16 * u + 128 * j)
            dref_v[0, pl.ds(16 * u, 16)] = jnp.where(
                pos < cntl, pos + offl_me, K + iota)
        pltpu.sync_copy(lk_v.at[pl.ds(128 * j, 128)], selk_s.at[dref_v.at[0]])
        pltpu.sync_copy(li_v.at[pl.ds(128 * j, 128)], seli_s.at[dref_v.at[0]])
        return 0
    lax.fori_loop(0, (cntl + 127) // 128, put_less, 0)

    def put_eq(j, _):
        for u in range(8):
            pos = iota + (16 * u + 128 * j)
            grank = pos + offe_me
            ok = (pos < cnte) & (grank < mneed)
            dref_v[0, pl.ds(16 * u, 16)] = jnp.where(ok, grank + L, K + iota)
        pltpu.sync_copy(skey_v.at[pl.ds(0, 128)], selk_s.at[dref_v.at[0]])
        pltpu.sync_copy(ei_v.at[pl.ds(128 * j, 128)], seli_s.at[dref_v.at[0]])
        return 0
    lax.fori_loop(0, (cnte + 127) // 128, put_eq, 0)
    plsc.subcore_barrier()

    @pl.when(t == 0)
    def _():
        pltpu.sync_copy(selk_s.at[pl.ds(0, K)], skey_v)
        pltpu.sync_copy(seli_s.at[pl.ds(0, K)], sidx_v)
        bufs = [(skey_v, sidx_v, skey2_v, sidx2_v),
                (skey2_v, sidx2_v, skey_v, sidx_v),
                (skey_v, sidx_v, skey2_v, sidx2_v)]
        for (sh, mask), (kb, ib, kb2, ib2) in zip(
                ((0, 0x7FF), (11, 0x7FF), (22, 0x3FF)), bufs):
            _clear_hist(VPC)

            def sph(vv, _):
                for u in range(4):
                    kv = plsc.bitcast(kb[pl.ds(64 * vv + 16 * u, 16)], u32)
                    d = ((kv >> u32(sh)) & u32(mask)).astype(i32)
                    rc, lastm = plsc.scan_count(d)
                    plsc.addupdate_scatter(hist_v, [d],
                                           rc.astype(i32) - bias + 1,
                                           mask=lastm)
                return 0
            lax.fori_loop(0, K // 64, sph, 0)

            def spx(vv, carry):
                for u in range(4):
                    o = 64 * vv + 16 * u
                    h = hist_v[pl.ds(o, 16)]
                    chs = plsc.cumsum(h)
                    hist2_v[pl.ds(o, 16)] = chs - h + carry
                    carry = carry + jnp.max(chs)
                return carry
            lax.fori_loop(0, VPC // 4, spx, i32(0))

            def sp(v, _):
                kv = plsc.bitcast(kb[pl.ds(16 * v, 16)], u32)
                d = ((kv >> u32(sh)) & u32(mask)).astype(i32)
                rc, lastm = plsc.scan_count(d)
                occ = rc.astype(i32) - bias
                base = plsc.load_gather(hist2_v, [d])
                dest = base + occ
                plsc.store_scatter(kb2, [dest], plsc.bitcast(kv, i32))
                plsc.store_scatter(ib2, [dest], ib[pl.ds(16 * v, 16)])
                plsc.addupdate_scatter(hist2_v, [d], occ + 1, mask=lastm)
                return 0
            lax.fori_loop(0, K // 16, sp, 0)

        @pl.when(c == 0)
        def _():
            pltpu.sync_copy(sidx2_v, out_hbm)


def _gather_body(idx_hbm, xflat_hbm, out_hbm, li_v, gidx_v, gbuf_v, sem):
    c = lax.axis_index("c")
    t = lax.axis_index("s")
    i32 = jnp.int32

    pltpu.sync_copy(idx_hbm, li_v.at[pl.ds(0, K)])
    my_r0 = c.astype(i32) * 64 + t * 4

    def fill_gidx(jj, _):
        rowbase = (my_r0 + jj // 8) * N
        for u in range(8):
            src = li_v[pl.ds(128 * (jj % 8) + 16 * u, 16)]
            gidx_v[jj, pl.ds(16 * u, 16)] = src + rowbase
        return 0
    lax.fori_loop(0, 32, fill_gidx, 0)

    def fire(jj, _):
        pltpu.async_copy(xflat_hbm.at[gidx_v.at[jj]],
                         gbuf_v.at[pl.ds(128 * jj, 128)], sem)
        return 0
    lax.fori_loop(0, 32, fire, 0)

    def drain(jj, _):
        pltpu.make_async_copy(xflat_hbm.at[gidx_v.at[jj]],
                              gbuf_v.at[pl.ds(128 * jj, 128)], sem).wait()
        return 0
    lax.fori_loop(0, 32, drain, 0)

    def wout(r4, _):
        pltpu.sync_copy(gbuf_v.at[pl.ds(K * r4, K)], out_hbm.at[my_r0 + r4])
        return 0
    lax.fori_loop(0, 4, wout, 0)


@jax.jit
def kernel(x, importance_scores):
    mesh = plsc.VectorSubcoreMesh(core_axis_name="c", subcore_axis_name="s")
    sel = pl.kernel(
        _topk_select_body,
        out_type=jax.ShapeDtypeStruct((K,), jnp.int32),
        mesh=mesh,
        compiler_params=pltpu.CompilerParams(needs_layout_passes=False),
        scratch_types=[
            pltpu.VMEM((CH,), jnp.float32),
            pltpu.VMEM((CH,), jnp.uint32),
            pltpu.VMEM((CH + 80,), jnp.uint32),
            pltpu.VMEM((CH + 80,), jnp.uint32),
            pltpu.VMEM((2048,), jnp.int32),
            pltpu.VMEM((2048,), jnp.int32),
            pltpu.VMEM((CH + 16,), jnp.uint32),
            pltpu.VMEM((CH + 16,), jnp.int32),
            pltpu.VMEM((CH + 16,), jnp.int32),
            pltpu.VMEM((1, 128), jnp.int32),
            pltpu.VMEM((16,), jnp.int32),
            pltpu.VMEM((16, 16), jnp.int32),
            pltpu.VMEM((K,), jnp.int32),
            pltpu.VMEM((K,), jnp.int32),
            pltpu.VMEM((K,), jnp.int32),
            pltpu.VMEM((K,), jnp.int32),
            pltpu.VMEM_SHARED((2048,), jnp.int32),
            pltpu.VMEM_SHARED((2048,), jnp.int32),
            pltpu.VMEM_SHARED((1024,), jnp.int32),
            pltpu.VMEM_SHARED((16, 16), jnp.int32),
            pltpu.VMEM_SHARED((16,), jnp.int32),
            pltpu.VMEM_SHARED((K + TRASH,), jnp.uint32),
            pltpu.VMEM_SHARED((K + TRASH,), jnp.int32),
        ],
    )
    gat = pl.kernel(
        _gather_body,
        out_type=jax.ShapeDtypeStruct((ROWS, K), jnp.float32),
        mesh=mesh,
        compiler_params=pltpu.CompilerParams(needs_layout_passes=False),
        scratch_types=[
            pltpu.VMEM((K,), jnp.int32),
            pltpu.VMEM((32, 128), jnp.int32),
            pltpu.VMEM((4 * K,), jnp.float32),
            pltpu.SemaphoreType.DMA,
        ],
    )
    idx = sel(importance_scores)
    return gat(idx, x.reshape(-1))


# speedup vs baseline: 1.0919x; 1.0919x over previous
import jax
import jax.numpy as jnp
from jax import lax
from jax.experimental import pallas as pl
from jax.experimental.pallas import tpu as pltpu
from jax.experimental.pallas import tpu_sc as plsc

N = 32768
K = 1024
ROWS = 128
NT = 16
CH = N // NT
VPC = CH // 16
TRASH = 16


def _topk_select_body(scores_hbm, out_hbm,
                      score_v, km_v, cand_v, cand2_v, hist_v, hist2_v,
                      lk_v, li_v, ei_v, dref_v, misc_v, cnt_v, skey_v,
                      sidx_v, skey2_v, sidx2_v,
                      shistA_s, shistB_s, shistC_s, cnts_s, bc_s, selk_s,
                      seli_s):
    c = lax.axis_index("c")
    t = lax.axis_index("s")
    iota = lax.iota(jnp.int32, 16)
    zero16 = jnp.zeros((16,), jnp.int32)
    i32 = jnp.int32
    u32 = jnp.uint32

    def _lane(v, i):
        return jnp.sum(jnp.where(iota == i, v, 0))

    def _clear_hist(nreg):
        def b(vv, _):
            for u in range(8):
                hist_v[pl.ds(128 * vv + 16 * u, 16)] = zero16
            return 0
        lax.fori_loop(0, nreg // 8, b, 0)

    rc0, _ = plsc.scan_count(zero16)
    bias = jnp.min(rc0.astype(i32))

    pltpu.sync_copy(scores_hbm.at[pl.ds(t * CH, CH)], score_v)
    _clear_hist(VPC)
    pltpu.sync_copy(hist_v.at[pl.ds(0, 128)], shistA_s.at[pl.ds(t * 128, 128)])
    pltpu.sync_copy(hist_v.at[pl.ds(0, 128)], shistB_s.at[pl.ds(t * 128, 128)])
    pltpu.sync_copy(hist_v.at[pl.ds(0, 64)], shistC_s.at[pl.ds(t * 64, 64)])

    def km_body(vv, _):
        for u in range(4):
            o = 64 * vv + 16 * u
            f = score_v[pl.ds(o, 16)]
            b = plsc.bitcast(f, u32)
            neg = (b & u32(0x80000000)) != u32(0)
            m = jnp.where(neg, ~b, b | u32(0x80000000))
            km_v[pl.ds(o, 16)] = ~m
        return 0
    lax.fori_loop(0, VPC // 4, km_body, 0)
    plsc.subcore_barrier()

    def local_hist(src_ref, nvreg4, shift, mask):
        def b(vv, _):
            for u in range(4):
                kv = src_ref[pl.ds(64 * vv + 16 * u, 16)]
                d = ((kv >> u32(shift)) & u32(mask)).astype(i32)
                rc, lastm = plsc.scan_count(d)
                plsc.addupdate_scatter(hist_v, [d], rc.astype(i32) - bias + 1,
                                       mask=lastm)
            return 0
        lax.fori_loop(0, nvreg4, b, 0)

    def scatter_add_hist(dst_s, nchunk):
        def b(j, _):
            for u in range(8):
                dref_v[0, pl.ds(16 * u, 16)] = iota + (16 * u + 128 * j)
            pltpu.sync_copy(hist_v.at[pl.ds(128 * j, 128)],
                            dst_s.at[dref_v.at[0]], add=True)
            return 0
        lax.fori_loop(0, nchunk, b, 0)

    def scan_and_publish(src_s, nvreg, target):
        pltpu.sync_copy(src_s, hist_v.at[pl.ds(0, 16 * nvreg)])

        def b(vv, carry):
            cum, vstar, cbef = carry
            for u in range(4):
                h = hist_v[pl.ds(64 * vv + 16 * u, 16)]
                s = jnp.sum(h)
                ncum = cum + s
                cross = (cum < target) & (ncum >= target)
                vstar = jnp.where(cross, 4 * vv + u, vstar)
                cbef = jnp.where(cross, cum, cbef)
                cum = ncum
            return (cum, vstar, cbef)
        _, vstar, cb = lax.fori_loop(0, nvreg // 4, b,
                                     (i32(0), i32(0), i32(0)))
        h = hist_v[pl.ds(16 * vstar, 16)]
        ch = plsc.cumsum(h) + cb
        hitv = ch >= target
        pos = jnp.max(plsc.all_reduce_ffs(hitv).astype(i32))
        bst = 16 * vstar + pos
        cbef = _lane(ch - h, pos)
        misc_v[...] = jnp.where(iota == 0, bst, jnp.where(iota == 1, cbef, 0))
        pltpu.sync_copy(misc_v, bc_s)

    def read_bc():
        pltpu.sync_copy(bc_s, misc_v)
        v = misc_v[...]
        return _lane(v, 0), _lane(v, 1)

    def compact_cands(src_ref, dst_ref, nvreg4, shift, mask, want):
        def b(vv, off):
            for u in range(4):
                kv = src_ref[pl.ds(64 * vv + 16 * u, 16)]
                mk = ((kv >> u32(shift)) & u32(mask)).astype(i32) == want
                plsc.store_compressed(dst_ref.at[pl.ds(off, 16)], kv, mask=mk)
                off = off + jnp.sum(mk.astype(i32))
            return off
        n = lax.fori_loop(0, nvreg4, b, i32(0))
        sent = jnp.full((16,), 0xFFFFFFFF, u32)
        for u in range(4):
            dst_ref[pl.ds(n + 16 * u, 16)] = sent
        return n

    _clear_hist(VPC)
    local_hist(km_v, VPC // 4, 21, 0x7FF)
    scatter_add_hist(shistA_s, 16)
    plsc.subcore_barrier()

    @pl.when(t == 0)
    def _():
        scan_and_publish(shistA_s, VPC, i32(K))
    plsc.subcore_barrier()
    b1, c1 = read_bc()
    r1 = i32(K - 1) - c1

    nc1 = compact_cands(km_v, cand_v, VPC // 4, 21, 0x7FF, b1)
    ncv1 = (nc1 + 63) // 64
    _clear_hist(VPC)
    local_hist(cand_v, ncv1, 10, 0x7FF)
    scatter_add_hist(shistB_s, 16)
    plsc.subcore_barrier()

    @pl.when(t == 0)
    def _():
        scan_and_publish(shistB_s, VPC, r1 + 1)
    plsc.subcore_barrier()
    b2, c2 = read_bc()
    r2 = r1 - c2

    nc2 = compact_cands(cand_v, cand2_v, ncv1, 10, 0x7FF, b2)
    ncv2 = (nc2 + 63) // 64
    _clear_hist(64)
    local_hist(cand2_v, ncv2, 0, 0x3FF)
    scatter_add_hist(shistC_s, 8)
    plsc.subcore_barrier()

    @pl.when(t == 0)
    def _():
        scan_and_publish(shistC_s, 64, r2 + 1)
    plsc.subcore_barrier()
    b3, c3 = read_bc()

    T = ((b1.astype(u32) << 21) | (b2.astype(u32) << 10) | b3.astype(u32))
    L = c1 + c2 + c3
    mneed = i32(K) - L

    def comp2(vv, carry):
        offl, offe = carry
        for u in range(4):
            o = 64 * vv + 16 * u
            kv = km_v[pl.ds(o, 16)]
            gi = iota + (o + CH * t)
            ml = kv < T
            me = kv == T
            plsc.store_compressed(lk_v.at[pl.ds(offl, 16)], kv, mask=ml)
            plsc.store_compressed(li_v.at[pl.ds(offl, 16)], gi, mask=ml)
            plsc.store_compressed(ei_v.at[pl.ds(offe, 16)], gi, mask=me)
            offl = offl + jnp.sum(ml.astype(i32))
            offe = offe + jnp.sum(me.astype(i32))
        return (offl, offe)
    cntl, cnte = lax.fori_loop(0, VPC // 4, comp2, (i32(0), i32(0)))

    misc_v[...] = jnp.where(iota == 0, cntl, jnp.where(iota == 1, cnte, 0))
    pltpu.sync_copy(misc_v, cnts_s.at[t])
    plsc.subcore_barrier()

    pltpu.sync_copy(cnts_s, cnt_v)
    lessc = plsc.load_gather(cnt_v, [iota, zero16])
    eqc = plsc.load_gather(cnt_v, [iota, zero16 + 1])
    cl = plsc.cumsum(lessc)
    ce = plsc.cumsum(eqc)
    offl_me = _lane(cl - lessc, t)
    offe_me = _lane(ce - eqc, t)

    tk = plsc.bitcast(jnp.zeros((16,), u32) + T, i32)
    for u in range(8):
        skey_v[pl.ds(16 * u, 16)] = tk

    def put_less(j, _):
        for u in range(8):
            pos = iota + (16 * u + 128 * j)
            dref_v[0, pl.ds(16 * u, 16)] = jnp.where(
                pos < cntl, pos + offl_me, K + iota)
        pltpu.sync_copy(lk_v.at[pl.ds(128 * j, 128)], selk_s.at[dref_v.at[0]])
        pltpu.sync_copy(li_v.at[pl.ds(128 * j, 128)], seli_s.at[dref_v.at[0]])
        return 0
    lax.fori_loop(0, (cntl + 127) // 128, put_less, 0)

    def put_eq(j, _):
        for u in range(8):
            pos = iota + (16 * u + 128 * j)
            grank = pos + offe_me
            ok = (pos < cnte) & (grank < mneed)
            dref_v[0, pl.ds(16 * u, 16)] = jnp.where(ok, grank + L, K + iota)
        pltpu.sync_copy(skey_v.at[pl.ds(0, 128)], selk_s.at[dref_v.at[0]])
        pltpu.sync_copy(ei_v.at[pl.ds(128 * j, 128)], seli_s.at[dref_v.at[0]])
        return 0
    lax.fori_loop(0, (cnte + 127) // 128, put_eq, 0)
    plsc.subcore_barrier()

    @pl.when(t == 0)
    def _():
        pltpu.sync_copy(selk_s.at[pl.ds(0, K)], skey_v)
        pltpu.sync_copy(seli_s.at[pl.ds(0, K)], sidx_v)
        bufs = [(skey_v, sidx_v, skey2_v, sidx2_v),
                (skey2_v, sidx2_v, skey_v, sidx_v),
                (skey_v, sidx_v, skey2_v, sidx2_v)]
        for (sh, mask), (kb, ib, kb2, ib2) in zip(
                ((0, 0x7FF), (11, 0x7FF), (22, 0x3FF)), bufs):
            _clear_hist(VPC)

            def sph(vv, _):
                for u in range(4):
                    kv = plsc.bitcast(kb[pl.ds(64 * vv + 16 * u, 16)], u32)
                    d = ((kv >> u32(sh)) & u32(mask)).astype(i32)
                    rc, lastm = plsc.scan_count(d)
                    plsc.addupdate_scatter(hist_v, [d],
                                           rc.astype(i32) - bias + 1,
                                           mask=lastm)
                return 0
            lax.fori_loop(0, K // 64, sph, 0)

            def spx(vv, carry):
                for u in range(4):
                    o = 64 * vv + 16 * u
                    h = hist_v[pl.ds(o, 16)]
                    chs = plsc.cumsum(h)
                    hist2_v[pl.ds(o, 16)] = chs - h + carry
                    carry = carry + jnp.max(chs)
                return carry
            lax.fori_loop(0, VPC // 4, spx, i32(0))

            def sp(v, _):
                kv = plsc.bitcast(kb[pl.ds(16 * v, 16)], u32)
                d = ((kv >> u32(sh)) & u32(mask)).astype(i32)
                rc, lastm = plsc.scan_count(d)
                occ = rc.astype(i32) - bias
                base = plsc.load_gather(hist2_v, [d])
                dest = base + occ
                plsc.store_scatter(kb2, [dest], plsc.bitcast(kv, i32))
                plsc.store_scatter(ib2, [dest], ib[pl.ds(16 * v, 16)])
                plsc.addupdate_scatter(hist2_v, [d], occ + 1, mask=lastm)
                return 0
            lax.fori_loop(0, K // 16, sp, 0)

        @pl.when(c == 0)
        def _():
            pltpu.sync_copy(sidx2_v, out_hbm)


def _gather_rows_body(idx_hbm, xt_hbm, outt_hbm, idx_v, rows_v, sem):
    c = lax.axis_index("c")
    t = lax.axis_index("s")
    wid = t * 2 + c
    base = wid * (K // 32)
    pltpu.sync_copy(idx_hbm.at[pl.ds(base, K // 32)], idx_v)
    pltpu.async_copy(xt_hbm.at[idx_v], rows_v, sem).wait()
    pltpu.sync_copy(rows_v, outt_hbm.at[pl.ds(base, K // 32)])


@jax.jit
def kernel(x, importance_scores):
    mesh = plsc.VectorSubcoreMesh(core_axis_name="c", subcore_axis_name="s")
    sel = pl.kernel(
        _topk_select_body,
        out_type=jax.ShapeDtypeStruct((K,), jnp.int32),
        mesh=mesh,
        compiler_params=pltpu.CompilerParams(needs_layout_passes=False),
        scratch_types=[
            pltpu.VMEM((CH,), jnp.float32),
            pltpu.VMEM((CH,), jnp.uint32),
            pltpu.VMEM((CH + 80,), jnp.uint32),
            pltpu.VMEM((CH + 80,), jnp.uint32),
            pltpu.VMEM((2048,), jnp.int32),
            pltpu.VMEM((2048,), jnp.int32),
            pltpu.VMEM((CH + 16,), jnp.uint32),
            pltpu.VMEM((CH + 16,), jnp.int32),
            pltpu.VMEM((CH + 16,), jnp.int32),
            pltpu.VMEM((1, 128), jnp.int32),
            pltpu.VMEM((16,), jnp.int32),
            pltpu.VMEM((16, 16), jnp.int32),
            pltpu.VMEM((K,), jnp.int32),
            pltpu.VMEM((K,), jnp.int32),
            pltpu.VMEM((K,), jnp.int32),
            pltpu.VMEM((K,), jnp.int32),
            pltpu.VMEM_SHARED((2048,), jnp.int32),
            pltpu.VMEM_SHARED((2048,), jnp.int32),
            pltpu.VMEM_SHARED((1024,), jnp.int32),
            pltpu.VMEM_SHARED((16, 16), jnp.int32),
            pltpu.VMEM_SHARED((16,), jnp.int32),
            pltpu.VMEM_SHARED((K + TRASH,), jnp.uint32),
            pltpu.VMEM_SHARED((K + TRASH,), jnp.int32),
        ],
    )
    gat = pl.kernel(
        _gather_rows_body,
        out_type=jax.ShapeDtypeStruct((K, ROWS), jnp.float32),
        mesh=mesh,
        compiler_params=pltpu.CompilerParams(needs_layout_passes=False),
        scratch_types=[
            pltpu.VMEM((K // 32,), jnp.int32),
            pltpu.VMEM((K // 32, ROWS), jnp.float32),
            pltpu.SemaphoreType.DMA,
        ],
    )
    idx = sel(importance_scores)
    outt = gat(idx, x.T)
    return outt.T


# speedup vs baseline: 1.1461x; 1.0497x over previous
import jax
import jax.numpy as jnp
from jax import lax
from jax.experimental import pallas as pl
from jax.experimental.pallas import tpu as pltpu
from jax.experimental.pallas import tpu_sc as plsc

N = 32768
K = 1024
ROWS = 128
NT = 16
CH = N // NT
VPC = CH // 16
TRASH = 16


def _topk_gather_body(scores_hbm, xt_hbm, outt_hbm,
                      score_v, km_v, cand_v, cand2_v, hist_v, hist2_v,
                      lk_v, li_v, ei_v, dref_v, misc_v, cnt_v, skey_v,
                      sidx_v, skey2_v, sidx2_v, idx_v, rows_v,
                      shistA_s, shistB_s, shistC_s, cnts_s, bc_s, selk_s,
                      seli_s, sout_s, sem):
    c = lax.axis_index("c")
    t = lax.axis_index("s")
    iota = lax.iota(jnp.int32, 16)
    zero16 = jnp.zeros((16,), jnp.int32)
    i32 = jnp.int32
    u32 = jnp.uint32

    def _lane(v, i):
        return jnp.sum(jnp.where(iota == i, v, 0))

    def _clear_hist(nreg):
        def b(vv, _):
            for u in range(8):
                hist_v[pl.ds(128 * vv + 16 * u, 16)] = zero16
            return 0
        lax.fori_loop(0, nreg // 8, b, 0)

    rc0, _ = plsc.scan_count(zero16)
    bias = jnp.min(rc0.astype(i32))

    pltpu.sync_copy(scores_hbm.at[pl.ds(t * CH, CH)], score_v)
    _clear_hist(VPC)
    pltpu.sync_copy(hist_v.at[pl.ds(0, 128)], shistA_s.at[pl.ds(t * 128, 128)])
    pltpu.sync_copy(hist_v.at[pl.ds(0, 128)], shistB_s.at[pl.ds(t * 128, 128)])
    pltpu.sync_copy(hist_v.at[pl.ds(0, 64)], shistC_s.at[pl.ds(t * 64, 64)])

    def km_body(vv, _):
        for u in range(4):
            o = 64 * vv + 16 * u
            f = score_v[pl.ds(o, 16)]
            b = plsc.bitcast(f, u32)
            neg = (b & u32(0x80000000)) != u32(0)
            m = jnp.where(neg, ~b, b | u32(0x80000000))
            km_v[pl.ds(o, 16)] = ~m
        return 0
    lax.fori_loop(0, VPC // 4, km_body, 0)
    plsc.subcore_barrier()

    def local_hist(src_ref, nvreg4, shift, mask):
        def b(vv, _):
            for u in range(4):
                kv = src_ref[pl.ds(64 * vv + 16 * u, 16)]
                d = ((kv >> u32(shift)) & u32(mask)).astype(i32)
                rc, lastm = plsc.scan_count(d)
                plsc.addupdate_scatter(hist_v, [d], rc.astype(i32) - bias + 1,
                                       mask=lastm)
            return 0
        lax.fori_loop(0, nvreg4, b, 0)

    def scatter_add_hist(dst_s, nchunk):
        def b(j, _):
            for u in range(8):
                dref_v[0, pl.ds(16 * u, 16)] = iota + (16 * u + 128 * j)
            pltpu.sync_copy(hist_v.at[pl.ds(128 * j, 128)],
                            dst_s.at[dref_v.at[0]], add=True)
            return 0
        lax.fori_loop(0, nchunk, b, 0)

    def scan_and_publish(src_s, nvreg, target):
        pltpu.sync_copy(src_s, hist_v.at[pl.ds(0, 16 * nvreg)])

        def b(vv, carry):
            cum, vstar, cbef = carry
            for u in range(4):
                h = hist_v[pl.ds(64 * vv + 16 * u, 16)]
                s = jnp.sum(h)
                ncum = cum + s
                cross = (cum < target) & (ncum >= target)
                vstar = jnp.where(cross, 4 * vv + u, vstar)
                cbef = jnp.where(cross, cum, cbef)
                cum = ncum
            return (cum, vstar, cbef)
        _, vstar, cb = lax.fori_loop(0, nvreg // 4, b,
                                     (i32(0), i32(0), i32(0)))
        h = hist_v[pl.ds(16 * vstar, 16)]
        ch = plsc.cumsum(h) + cb
        hitv = ch >= target
        pos = jnp.max(plsc.all_reduce_ffs(hitv).astype(i32))
        bst = 16 * vstar + pos
        cbef = _lane(ch - h, pos)
        misc_v[...] = jnp.where(iota == 0, bst, jnp.where(iota == 1, cbef, 0))
        pltpu.sync_copy(misc_v, bc_s)

    def read_bc():
        pltpu.sync_copy(bc_s, misc_v)
        v = misc_v[...]
        return _lane(v, 0), _lane(v, 1)

    def compact_cands(src_ref, dst_ref, nvreg4, shift, mask, want):
        def b(vv, off):
            for u in range(4):
                kv = src_ref[pl.ds(64 * vv + 16 * u, 16)]
                mk = ((kv >> u32(shift)) & u32(mask)).astype(i32) == want
                plsc.store_compressed(dst_ref.at[pl.ds(off, 16)], kv, mask=mk)
                off = off + jnp.sum(mk.astype(i32))
            return off
        n = lax.fori_loop(0, nvreg4, b, i32(0))
        sent = jnp.full((16,), 0xFFFFFFFF, u32)
        for u in range(4):
            dst_ref[pl.ds(n + 16 * u, 16)] = sent
        return n

    _clear_hist(VPC)
    local_hist(km_v, VPC // 4, 21, 0x7FF)
    scatter_add_hist(shistA_s, 16)
    plsc.subcore_barrier()

    @pl.when(t == 0)
    def _():
        scan_and_publish(shistA_s, VPC, i32(K))
    plsc.subcore_barrier()
    b1, c1 = read_bc()
    r1 = i32(K - 1) - c1

    nc1 = compact_cands(km_v, cand_v, VPC // 4, 21, 0x7FF, b1)
    ncv1 = (nc1 + 63) // 64
    _clear_hist(VPC)
    local_hist(cand_v, ncv1, 10, 0x7FF)
    scatter_add_hist(shistB_s, 16)
    plsc.subcore_barrier()

    @pl.when(t == 0)
    def _():
        scan_and_publish(shistB_s, VPC, r1 + 1)
    plsc.subcore_barrier()
    b2, c2 = read_bc()
    r2 = r1 - c2

    nc2 = compact_cands(cand_v, cand2_v, ncv1, 10, 0x7FF, b2)
    ncv2 = (nc2 + 63) // 64
    _clear_hist(64)
    local_hist(cand2_v, ncv2, 0, 0x3FF)
    scatter_add_hist(shistC_s, 8)
    plsc.subcore_barrier()

    @pl.when(t == 0)
    def _():
        scan_and_publish(shistC_s, 64, r2 + 1)
    plsc.subcore_barrier()
    b3, c3 = read_bc()

    T = ((b1.astype(u32) << 21) | (b2.astype(u32) << 10) | b3.astype(u32))
    L = c1 + c2 + c3
    mneed = i32(K) - L

    def comp2(vv, carry):
        offl, offe = carry
        for u in range(4):
            o = 64 * vv + 16 * u
            kv = km_v[pl.ds(o, 16)]
            gi = iota + (o + CH * t)
            ml = kv < T
            me = kv == T
            plsc.store_compressed(lk_v.at[pl.ds(offl, 16)], kv, mask=ml)
            plsc.store_compressed(li_v.at[pl.ds(offl, 16)], gi, mask=ml)
            plsc.store_compressed(ei_v.at[pl.ds(offe, 16)], gi, mask=me)
            offl = offl + jnp.sum(ml.astype(i32))
            offe = offe + jnp.sum(me.astype(i32))
        return (offl, offe)
    cntl, cnte = lax.fori_loop(0, VPC // 4, comp2, (i32(0), i32(0)))

    misc_v[...] = jnp.where(iota == 0, cntl, jnp.where(iota == 1, cnte, 0))
    pltpu.sync_copy(misc_v, cnts_s.at[t])
    plsc.subcore_barrier()

    pltpu.sync_copy(cnts_s, cnt_v)
    lessc = plsc.load_gather(cnt_v, [iota, zero16])
    eqc = plsc.load_gather(cnt_v, [iota, zero16 + 1])
    cl = plsc.cumsum(lessc)
    ce = plsc.cumsum(eqc)
    offl_me = _lane(cl - lessc, t)
    offe_me = _lane(ce - eqc, t)

    tk = plsc.bitcast(jnp.zeros((16,), u32) + T, i32)
    for u in range(8):
        skey_v[pl.ds(16 * u, 16)] = tk

    def put_less(j, _):
        for u in range(8):
            pos = iota + (16 * u + 128 * j)
            dref_v[0, pl.ds(16 * u, 16)] = jnp.where(
                pos < cntl, pos + offl_me, K + iota)
        pltpu.sync_copy(lk_v.at[pl.ds(128 * j, 128)], selk_s.at[dref_v.at[0]])
        pltpu.sync_copy(li_v.at[pl.ds(128 * j, 128)], seli_s.at[dref_v.at[0]])
        return 0
    lax.fori_loop(0, (cntl + 127) // 128, put_less, 0)

    def put_eq(j, _):
        for u in range(8):
            pos = iota + (16 * u + 128 * j)
            grank = pos + offe_me
            ok = (pos < cnte) & (grank < mneed)
            dref_v[0, pl.ds(16 * u, 16)] = jnp.where(ok, grank + L, K + iota)
        pltpu.sync_copy(skey_v.at[pl.ds(0, 128)], selk_s.at[dref_v.at[0]])
        pltpu.sync_copy(ei_v.at[pl.ds(128 * j, 128)], seli_s.at[dref_v.at[0]])
        return 0
    lax.fori_loop(0, (cnte + 127) // 128, put_eq, 0)
    plsc.subcore_barrier()

    @pl.when(t == 0)
    def _():
        pltpu.sync_copy(selk_s.at[pl.ds(0, K)], skey_v)
        pltpu.sync_copy(seli_s.at[pl.ds(0, K)], sidx_v)
        bufs = [(skey_v, sidx_v, skey2_v, sidx2_v),
                (skey2_v, sidx2_v, skey_v, sidx_v),
                (skey_v, sidx_v, skey2_v, sidx2_v)]
        for (sh, mask), (kb, ib, kb2, ib2) in zip(
                ((0, 0x7FF), (11, 0x7FF), (22, 0x3FF)), bufs):
            _clear_hist(VPC)

            def sph(vv, _):
                for u in range(4):
                    kv = plsc.bitcast(kb[pl.ds(64 * vv + 16 * u, 16)], u32)
                    d = ((kv >> u32(sh)) & u32(mask)).astype(i32)
                    rc, lastm = plsc.scan_count(d)
                    plsc.addupdate_scatter(hist_v, [d],
                                           rc.astype(i32) - bias + 1,
                                           mask=lastm)
                return 0
            lax.fori_loop(0, K // 64, sph, 0)

            def spx(vv, carry):
                for u in range(4):
                    o = 64 * vv + 16 * u
                    h = hist_v[pl.ds(o, 16)]
                    chs = plsc.cumsum(h)
                    hist2_v[pl.ds(o, 16)] = chs - h + carry
                    carry = carry + jnp.max(chs)
                return carry
            lax.fori_loop(0, VPC // 4, spx, i32(0))

            def sp(v, _):
                kv = plsc.bitcast(kb[pl.ds(16 * v, 16)], u32)
                d = ((kv >> u32(sh)) & u32(mask)).astype(i32)
                rc, lastm = plsc.scan_count(d)
                occ = rc.astype(i32) - bias
                base = plsc.load_gather(hist2_v, [d])
                dest = base + occ
                plsc.store_scatter(kb2, [dest], plsc.bitcast(kv, i32))
                plsc.store_scatter(ib2, [dest], ib[pl.ds(16 * v, 16)])
                plsc.addupdate_scatter(hist2_v, [d], occ + 1, mask=lastm)
                return 0
            lax.fori_loop(0, K // 16, sp, 0)

        pltpu.sync_copy(sidx2_v, sout_s)
    plsc.subcore_barrier()

    wid = c * NT + t
    base = wid * (K // 32)
    pltpu.sync_copy(sout_s.at[pl.ds(base, K // 32)], idx_v)
    pltpu.async_copy(xt_hbm.at[idx_v], rows_v, sem).wait()
    pltpu.sync_copy(rows_v, outt_hbm.at[pl.ds(base, K // 32)])


@jax.jit
def kernel(x, importance_scores):
    mesh = plsc.VectorSubcoreMesh(core_axis_name="c", subcore_axis_name="s")
    run = pl.kernel(
        _topk_gather_body,
        out_type=jax.ShapeDtypeStruct((K, ROWS), jnp.float32),
        mesh=mesh,
        compiler_params=pltpu.CompilerParams(needs_layout_passes=False),
        scratch_types=[
            pltpu.VMEM((CH,), jnp.float32),
            pltpu.VMEM((CH,), jnp.uint32),
            pltpu.VMEM((CH + 80,), jnp.uint32),
            pltpu.VMEM((CH + 80,), jnp.uint32),
            pltpu.VMEM((2048,), jnp.int32),
            pltpu.VMEM((2048,), jnp.int32),
            pltpu.VMEM((CH + 16,), jnp.uint32),
            pltpu.VMEM((CH + 16,), jnp.int32),
            pltpu.VMEM((CH + 16,), jnp.int32),
            pltpu.VMEM((1, 128), jnp.int32),
            pltpu.VMEM((16,), jnp.int32),
            pltpu.VMEM((16, 16), jnp.int32),
            pltpu.VMEM((K,), jnp.int32),
            pltpu.VMEM((K,), jnp.int32),
            pltpu.VMEM((K,), jnp.int32),
            pltpu.VMEM((K,), jnp.int32),
            pltpu.VMEM((K // 32,), jnp.int32),
            pltpu.VMEM((K // 32, ROWS), jnp.float32),
            pltpu.VMEM_SHARED((2048,), jnp.int32),
            pltpu.VMEM_SHARED((2048,), jnp.int32),
            pltpu.VMEM_SHARED((1024,), jnp.int32),
            pltpu.VMEM_SHARED((16, 16), jnp.int32),
            pltpu.VMEM_SHARED((16,), jnp.int32),
            pltpu.VMEM_SHARED((K + TRASH,), jnp.uint32),
            pltpu.VMEM_SHARED((K + TRASH,), jnp.int32),
            pltpu.VMEM_SHARED((K,), jnp.int32),
            pltpu.SemaphoreType.DMA,
        ],
    )
    outt = run(importance_scores, x.T)
    return outt.T
